# Initial kernel scaffold; baseline (speedup 1.0000x reference)
import jax
import jax.numpy as jnp
from jax.experimental import pallas as pl


def kernel(pos, edge_index, epsilon, sigma):
    raise NotImplementedError("write your pallas kernel here")


# speedup vs baseline: 40.8026x; 40.8026x over previous
import functools

import jax
import jax.numpy as jnp
from jax import lax
from jax.experimental import pallas as pl
from jax.experimental.pallas import tpu as pltpu
from jax.experimental.pallas import tpu_sc as plsc

NC = 2
NS = 16
NW = NC * NS
L = 16

CHUNK = 1024
SUB = 128
GROUPS = CHUNK // L


def _sc_body(n_nodes, n_chunks, iters_per_tile,
             pos_hbm, src_hbm, dst_hbm, par_hbm, out_hbm,
             src_idx_v, dst_idx_v, src_rows_v, dst_rows_v, par_v, acc_v, sem):
    wid = lax.axis_index("s") * NC + lax.axis_index("c")

    pltpu.sync_copy(par_hbm, par_v)
    s2v = par_v[pl.ds(0, L)]
    tev = par_v[pl.ds(L, L)]

    zero16 = jnp.zeros((L,), jnp.float32)

    def _zero(i, carry):
        acc_v[pl.ds(i * L, L)] = zero16
        return carry
    lax.fori_loop(0, n_nodes // L, _zero, 0)

    iota = lax.iota(jnp.int32, L)
    k0 = jnp.zeros((L,), jnp.int32)
    k1 = jnp.full((L,), 1, jnp.int32)
    k2 = jnp.full((L,), 2, jnp.int32)
    src_rows_2d = src_rows_v
    dst_rows_2d = dst_rows_v

    def _chunk(i, carry):
        cid = wid + i * NW

        @pl.when(cid < n_chunks)
        def _():
            ebase = cid * CHUNK
            pltpu.sync_copy(src_hbm.at[pl.ds(ebase, CHUNK)], src_idx_v)
            pltpu.sync_copy(dst_hbm.at[pl.ds(ebase, CHUNK)], dst_idx_v)
            copies = []
            for j in range(CHUNK // SUB):
                sl = pl.ds(j * SUB, SUB)
                copies.append(pltpu.async_copy(
                    pos_hbm.at[src_idx_v.at[sl]], src_rows_2d.at[sl], sem))
                copies.append(pltpu.async_copy(
                    pos_hbm.at[dst_idx_v.at[sl]], dst_rows_2d.at[sl], sem))
            for cp in copies:
                cp.wait()

            def _group(g, c):
                rows = g * L + iota
                sx = plsc.load_gather(src_rows_v, [rows, k0])
                sy = plsc.load_gather(src_rows_v, [rows, k1])
                sz = plsc.load_gather(src_rows_v, [rows, k2])
                dx = plsc.load_gather(dst_rows_v, [rows, k0])
                dy = plsc.load_gather(dst_rows_v, [rows, k1])
                dz = plsc.load_gather(dst_rows_v, [rows, k2])
                vx = dx - sx
                vy = dy - sy
                vz = dz - sz
                r2 = vx * vx + vy * vy + vz * vz
                t = s2v / r2
                x3 = t * t * t
                e = tev * (x3 * x3 - x3)
                sidx = src_idx_v[pl.ds(g * L, L)]
                plsc.addupdate_scatter(acc_v, [sidx], e)
                return c
            lax.fori_loop(0, GROUPS, _group, 0)
        return carry

    lax.fori_loop(0, iters_per_tile, _chunk, 0)
    pltpu.sync_copy(acc_v, out_hbm.at[wid])


def _tc_reduce_body(p_ref, o_ref):
    o_ref[...] = jnp.sum(p_ref[...], axis=0)


def kernel(pos, edge_index, epsilon, sigma):
    n_nodes = pos.shape[0]
    n_edges = edge_index.shape[1]
    assert n_edges % CHUNK == 0 and n_nodes % L == 0
    n_chunks = n_edges // CHUNK
    iters_per_tile = (n_chunks + NW - 1) // NW

    pos4 = jnp.pad(pos.astype(jnp.float32), ((0, 0), (0, 1)))
    src = edge_index[0]
    dst = edge_index[1]
    eps32 = jnp.asarray(epsilon, jnp.float32)
    sig32 = jnp.asarray(sigma, jnp.float32)
    par = jnp.concatenate([
        jnp.broadcast_to(sig32 * sig32, (L,)),
        jnp.broadcast_to(2.0 * eps32, (L,)),
    ])

    sc_fn = pl.kernel(
        functools.partial(_sc_body, n_nodes, n_chunks, iters_per_tile),
        out_type=jax.ShapeDtypeStruct((NW, n_nodes), jnp.float32),
        mesh=plsc.VectorSubcoreMesh(
            core_axis_name="c", subcore_axis_name="s",
            num_cores=NC, num_subcores=NS),
        scratch_types=[
            pltpu.VMEM((CHUNK,), jnp.int32),
            pltpu.VMEM((CHUNK,), jnp.int32),
            pltpu.VMEM((CHUNK, 4), jnp.float32),
            pltpu.VMEM((CHUNK, 4), jnp.float32),
            pltpu.VMEM((2 * L,), jnp.float32),
            pltpu.VMEM((n_nodes,), jnp.float32),
            pltpu.SemaphoreType.DMA,
        ],
        compiler_params=pltpu.CompilerParams(
            needs_layout_passes=False, use_tc_tiling_on_sc=False),
    )
    partials = sc_fn(pos4, src, dst, par)

    bc = 12288
    out = pl.pallas_call(
        _tc_reduce_body,
        grid=(pl.cdiv(n_nodes, bc),),
        in_specs=[pl.BlockSpec((NW, bc), lambda i: (0, i))],
        out_specs=pl.BlockSpec((bc,), lambda i: (i,)),
        out_shape=jax.ShapeDtypeStruct((n_nodes,), jnp.float32),
    )(partials)
    return out
